# baseline (speedup 1.0000x reference)
import jax
import jax.numpy as jnp
from jax.experimental import pallas as pl
from jax.experimental.pallas import tpu as pltpu

_R = 2048
_KC = 1024
_D = 2048
_E = 64


def _router_block(x_ref, w_ref, b_ref, idx_ref, probs_ref, acc_ref):
    j = pl.program_id(1)
    nk = pl.num_programs(1)
    partial = jnp.dot(x_ref[...], w_ref[...], preferred_element_type=jnp.float32)

    @pl.when(j == 0)
    def _():
        acc_ref[...] = partial

    @pl.when(j != 0)
    def _():
        acc_ref[...] += partial

    @pl.when(j == nk - 1)
    def _():
        logits = acc_ref[...] + b_ref[...]
        r, e = logits.shape
        col = jax.lax.broadcasted_iota(jnp.int32, (r, e), 1)
        m1 = jnp.max(logits, axis=1, keepdims=True)
        i1 = jnp.min(jnp.where(logits == m1, col, e), axis=1, keepdims=True)
        masked = jnp.where(col == i1, -jnp.inf, logits)
        m2 = jnp.max(masked, axis=1, keepdims=True)
        i2 = jnp.min(jnp.where(masked == m2, col, e), axis=1, keepdims=True)
        idx_ref[...] = jnp.concatenate([i1, i2], axis=1)
        ex = jnp.exp(m2 - m1)
        den = 1.0 + ex
        probs_ref[...] = jnp.concatenate([1.0 / den, ex / den], axis=1)


def kernel(x, W_gate, b_gate):
    n, d = x.shape
    e = W_gate.shape[1]
    idx, probs = pl.pallas_call(
        _router_block,
        grid=(n // _R, d // _KC),
        in_specs=[
            pl.BlockSpec((_R, _KC), lambda i, j: (i, j)),
            pl.BlockSpec((_KC, e), lambda i, j: (j, 0)),
            pl.BlockSpec((1, e), lambda i, j: (0, 0)),
        ],
        out_specs=[
            pl.BlockSpec((_R, 2), lambda i, j: (i, 0)),
            pl.BlockSpec((_R, 2), lambda i, j: (i, 0)),
        ],
        out_shape=[
            jax.ShapeDtypeStruct((n, 2), jnp.int32),
            jax.ShapeDtypeStruct((n, 2), jnp.float32),
        ],
        scratch_shapes=[pltpu.VMEM((_R, e), jnp.float32)],
        compiler_params=pltpu.CompilerParams(
            dimension_semantics=("arbitrary", "arbitrary"),
        ),
    )(x, W_gate, b_gate.reshape(1, e))
    return (idx, probs)
